# Initial kernel scaffold; baseline (speedup 1.0000x reference)
import jax
import jax.numpy as jnp
from jax.experimental import pallas as pl


def kernel(x1, edge_index1, x2, edge_index2, W1, b1, W2, b2, Wp1, bp1, Wp2, bp2, Wf1, bf1, Wf2, bf2, Wo, bo):
    raise NotImplementedError("write your pallas kernel here")


# speedup vs baseline: 22.1278x; 22.1278x over previous
import functools

import jax
import jax.numpy as jnp
from jax import lax
from jax.experimental import pallas as pl
from jax.experimental.pallas import tpu as pltpu
from jax.experimental.pallas import tpu_sc as plsc

N = 10000
D = 128
E = 320000
NP = 10240
EP = 327680
ROWS = EP // 128
NC, NS = 2, 16
RPT = ROWS // NS
CH = 40
NPT = NP // NS
NB = 512
GB = NP // NB

_mesh = plsc.VectorSubcoreMesh(
    core_axis_name="c", subcore_axis_name="s", num_cores=NC, num_subcores=NS)


def _leaky(x):
    return jnp.where(x >= 0, x, 0.01 * x)


@functools.partial(
    pl.kernel,
    out_type=jax.ShapeDtypeStruct((NC, NP, 128), jnp.float32),
    mesh=_mesh,
    scratch_types=[
        pltpu.VMEM_SHARED((NP, 128), jnp.float32),
        pltpu.VMEM((CH, 128), jnp.int32),
        pltpu.VMEM((128, 128), jnp.float32),
    ],
)
def _deg_kernel(dsts, ones_hbm, zdeg_hbm, deg_out, deg_sp, dst_i, ones_v):
    c = lax.axis_index("c")
    s = lax.axis_index("s")
    pltpu.sync_copy(zdeg_hbm.at[pl.ds(s * NPT, NPT)],
                    deg_sp.at[pl.ds(s * NPT, NPT)])
    pltpu.sync_copy(ones_hbm, ones_v)
    plsc.subcore_barrier()

    def chunk(b, carry):
        base = s * RPT + b * CH
        pltpu.sync_copy(dsts.at[c, pl.ds(base, CH)], dst_i)

        def body(j, inner):
            pltpu.sync_copy(ones_v, deg_sp.at[dst_i.at[j]], add=True)
            return inner

        lax.fori_loop(0, CH, body, 0)
        return carry

    lax.fori_loop(0, RPT // CH, chunk, 0)
    plsc.subcore_barrier()
    pltpu.sync_copy(deg_sp.at[pl.ds(s * NPT, NPT)],
                    deg_out.at[c, pl.ds(s * NPT, NPT)])


@functools.partial(
    pl.kernel,
    out_type=jax.ShapeDtypeStruct((NC, NP, D), jnp.float32),
    mesh=_mesh,
    scratch_types=[
        pltpu.VMEM_SHARED((NP, D), jnp.float32),
        pltpu.VMEM((CH, 128), jnp.int32),
        pltpu.VMEM((CH, 128), jnp.int32),
        pltpu.VMEM((128, D), jnp.float32),
        pltpu.SemaphoreType.DMA,
    ],
)
def _scatter_kernel(gflat, srcs, dsts, s_out, s_sp, src_i, dst_i, rows, sem):
    c = lax.axis_index("c")
    s = lax.axis_index("s")
    pltpu.sync_copy(gflat.at[pl.ds(c * NP + s * NPT, NPT)],
                    s_sp.at[pl.ds(s * NPT, NPT)])
    plsc.subcore_barrier()

    def chunk(b, carry):
        base = s * RPT + b * CH
        pltpu.sync_copy(srcs.at[c, pl.ds(base, CH)], src_i)
        pltpu.sync_copy(dsts.at[c, pl.ds(base, CH)], dst_i)

        def body(j, inner):
            pltpu.async_copy(gflat.at[src_i.at[j]], rows, sem).wait()
            pltpu.sync_copy(rows, s_sp.at[dst_i.at[j]], add=True)
            return inner

        lax.fori_loop(0, CH, body, 0)
        return carry

    lax.fori_loop(0, RPT // CH, chunk, 0)
    plsc.subcore_barrier()
    pltpu.sync_copy(s_sp.at[pl.ds(s * NPT, NPT)],
                    s_out.at[c, pl.ds(s * NPT, NPT)])


def _gmat_body(x_ref, w_ref, deg_ref, g_ref):
    deg = deg_ref[0][:, :1]
    dis = lax.rsqrt(deg + 1.0)
    h = jnp.dot(x_ref[0], w_ref[0], preferred_element_type=jnp.float32)
    g_ref[0] = h * dis


_gmat = pl.pallas_call(
    _gmat_body,
    grid=(NC, GB),
    in_specs=[
        pl.BlockSpec((1, NB, D), lambda g, i: (g, i, 0)),
        pl.BlockSpec((1, D, D), lambda g, i: (g, 0, 0)),
        pl.BlockSpec((1, NB, 128), lambda g, i: (g, i, 0)),
    ],
    out_specs=pl.BlockSpec((1, NB, D), lambda g, i: (g, i, 0)),
    out_shape=jax.ShapeDtypeStruct((NC, NP, D), jnp.float32),
    compiler_params=pltpu.CompilerParams(
        dimension_semantics=("arbitrary", "arbitrary")),
)


def _head_body(s_ref, deg_ref, b_ref, wp_ref, bp_ref, wf1_ref, bf1_ref,
               wf2_ref, bf2_ref, wo_ref, bo_ref, out_ref, acc_ref):
    i = pl.program_id(0)
    rem = N - i * NB
    mask = lax.broadcasted_iota(jnp.int32, (NB, D), 0) < rem
    reds = []
    for g in (0, 1):
        deg = deg_ref[g][:, :1]
        dis = lax.rsqrt(deg + 1.0)
        v = _leaky(dis * s_ref[g] + b_ref[g])
        v = jnp.where(mask, v, 0.0)
        reds.append(jnp.sum(v, axis=0, keepdims=True))

    @pl.when(i == 0)
    def _():
        acc_ref[0:1] = reds[0]
        acc_ref[1:2] = reds[1]

    @pl.when(i > 0)
    def _():
        acc_ref[0:1] += reds[0]
        acc_ref[1:2] += reds[1]

    @pl.when(i == GB - 1)
    def _():
        m1 = acc_ref[0:1] * (1.0 / N)
        m2 = acc_ref[1:2] * (1.0 / N)
        h1 = _leaky(jnp.dot(m1, wp_ref[0], preferred_element_type=jnp.float32)
                    + bp_ref[0])
        h2 = _leaky(jnp.dot(m2, wp_ref[1], preferred_element_type=jnp.float32)
                    + bp_ref[1])
        t = _leaky(jnp.dot(h1, wf1_ref[:D], preferred_element_type=jnp.float32)
                   + jnp.dot(h2, wf1_ref[D:], preferred_element_type=jnp.float32)
                   + bf1_ref[...])
        t = _leaky(jnp.dot(t, wf2_ref[...], preferred_element_type=jnp.float32)
                   + bf2_ref[...])
        o = jnp.dot(t, wo_ref[...], preferred_element_type=jnp.float32) + bo_ref[...]
        out_ref[...] = jax.nn.sigmoid(o)


_head = pl.pallas_call(
    _head_body,
    grid=(GB,),
    in_specs=[
        pl.BlockSpec((NC, NB, D), lambda i: (0, i, 0)),
        pl.BlockSpec((NC, NB, 128), lambda i: (0, i, 0)),
        pl.BlockSpec((NC, 1, D), lambda i: (0, 0, 0)),
        pl.BlockSpec((NC, D, D), lambda i: (0, 0, 0)),
        pl.BlockSpec((NC, 1, D), lambda i: (0, 0, 0)),
        pl.BlockSpec((2 * D, 256), lambda i: (0, 0)),
        pl.BlockSpec((1, 256), lambda i: (0, 0)),
        pl.BlockSpec((256, 64), lambda i: (0, 0)),
        pl.BlockSpec((1, 64), lambda i: (0, 0)),
        pl.BlockSpec((64, 1), lambda i: (0, 0)),
        pl.BlockSpec((1, 1), lambda i: (0, 0)),
    ],
    out_specs=pl.BlockSpec((1, 1), lambda i: (0, 0)),
    out_shape=jax.ShapeDtypeStruct((1, 1), jnp.float32),
    scratch_shapes=[pltpu.VMEM((2, D), jnp.float32)],
    compiler_params=pltpu.CompilerParams(
        dimension_semantics=("arbitrary",)),
)


def kernel(x1, edge_index1, x2, edge_index2, W1, b1, W2, b2, Wp1, bp1,
           Wp2, bp2, Wf1, bf1, Wf2, bf2, Wo, bo):
    pad = EP - E
    pad_src = (jnp.arange(pad, dtype=jnp.int32) % 128)
    pad_dst = N + (jnp.arange(pad, dtype=jnp.int32) % (NP - N))
    srcs = jnp.stack([
        jnp.concatenate([edge_index1[0], pad_src]),
        jnp.concatenate([edge_index2[0], pad_src]) + NP,
    ]).reshape(NC, ROWS, 128)
    dsts = jnp.stack([
        jnp.concatenate([edge_index1[1], pad_dst]),
        jnp.concatenate([edge_index2[1], pad_dst]),
    ]).reshape(NC, ROWS, 128)

    x_st = jnp.zeros((NC, NP, D), jnp.float32)
    x_st = x_st.at[0, :N].set(x1).at[1, :N].set(x2)
    w_st = jnp.stack([W1, W2])

    ones_hbm = jnp.ones((128, 128), jnp.float32)
    zdeg_hbm = jnp.zeros((NP, 128), jnp.float32)

    deg2d = _deg_kernel(dsts, ones_hbm, zdeg_hbm)
    gmat = _gmat(x_st, w_st, deg2d)
    smat = _scatter_kernel(gmat.reshape(NC * NP, D), srcs, dsts)

    b_st = jnp.stack([b1, b2]).reshape(NC, 1, D)
    wp_st = jnp.stack([Wp1, Wp2])
    bp_st = jnp.stack([bp1, bp2]).reshape(NC, 1, D)
    return _head(smat, deg2d, b_st, wp_st, bp_st, Wf1, bf1.reshape(1, 256),
                 Wf2, bf2.reshape(1, 64), Wo, bo.reshape(1, 1))


# speedup vs baseline: 33.0650x; 1.4943x over previous
import functools

import jax
import jax.numpy as jnp
from jax import lax
from jax.experimental import pallas as pl
from jax.experimental.pallas import tpu as pltpu
from jax.experimental.pallas import tpu_sc as plsc

N = 10000
D = 128
E = 320000
NP = 10240
EP = 327680
ROWS = EP // 128
NC, NS = 2, 16
RPT = ROWS // NS
CH = 40
NPT = NP // NS
NB = 512
GB = NP // NB

_mesh = plsc.VectorSubcoreMesh(
    core_axis_name="c", subcore_axis_name="s", num_cores=NC, num_subcores=NS)


def _leaky(x):
    return jnp.where(x >= 0, x, 0.01 * x)


DL = 64


@functools.partial(
    pl.kernel,
    out_type=jax.ShapeDtypeStruct((NC, NP, DL), jnp.float32),
    mesh=_mesh,
    scratch_types=[
        pltpu.VMEM_SHARED((NP, DL), jnp.float32),
        pltpu.VMEM((CH, 128), jnp.int32),
        pltpu.VMEM((128, DL), jnp.float32),
    ],
)
def _deg_kernel(dsts, ones_hbm, zdeg_hbm, deg_out, deg_sp, dst_i, ones_v):
    c = lax.axis_index("c")
    s = lax.axis_index("s")
    pltpu.sync_copy(zdeg_hbm.at[pl.ds(s * NPT, NPT)],
                    deg_sp.at[pl.ds(s * NPT, NPT)])
    pltpu.sync_copy(ones_hbm, ones_v)
    plsc.subcore_barrier()

    def chunk(b, carry):
        base = s * RPT + b * CH
        pltpu.sync_copy(dsts.at[c, pl.ds(base, CH)], dst_i)

        def body(j, inner):
            pltpu.sync_copy(ones_v, deg_sp.at[dst_i.at[j]], add=True)
            return inner

        lax.fori_loop(0, CH, body, 0)
        return carry

    lax.fori_loop(0, RPT // CH, chunk, 0)
    plsc.subcore_barrier()
    pltpu.sync_copy(deg_sp.at[pl.ds(s * NPT, NPT)],
                    deg_out.at[c, pl.ds(s * NPT, NPT)])


@functools.partial(
    pl.kernel,
    out_type=jax.ShapeDtypeStruct((NC, NP, D), jnp.float32),
    mesh=_mesh,
    scratch_types=[
        pltpu.VMEM_SHARED((NP, D), jnp.float32),
        pltpu.VMEM((CH, 128), jnp.int32),
        pltpu.VMEM((CH, 128), jnp.int32),
        pltpu.VMEM((128, D), jnp.float32),
        pltpu.VMEM((128, D), jnp.float32),
        pltpu.SemaphoreType.DMA,
        pltpu.SemaphoreType.DMA,
    ],
)
def _scatter_kernel(gflat, srcs, dsts, s_out, s_sp, src_i, dst_i,
                    rows0, rows1, sem0, sem1):
    c = lax.axis_index("c")
    s = lax.axis_index("s")
    dummy = gflat.at[pl.ds(0, 128)]

    def wait0():
        pltpu.make_async_copy(dummy, rows0, sem0).wait()

    def wait1():
        pltpu.make_async_copy(dummy, rows1, sem1).wait()

    pltpu.sync_copy(gflat.at[pl.ds(c * NP + s * NPT, NPT)],
                    s_sp.at[pl.ds(s * NPT, NPT)])
    plsc.subcore_barrier()

    def stage(b):
        base = s * RPT + b * CH
        pltpu.sync_copy(srcs.at[c, pl.ds(base, CH)], src_i)
        pltpu.sync_copy(dsts.at[c, pl.ds(base, CH)], dst_i)

    def pair(i, inner):
        j = 2 * i
        pltpu.async_copy(gflat.at[src_i.at[j + 1]], rows1, sem1)
        wait0()
        pltpu.sync_copy(rows0, s_sp.at[dst_i.at[j]], add=True)
        pltpu.async_copy(gflat.at[src_i.at[j + 2]], rows0, sem0)
        wait1()
        pltpu.sync_copy(rows1, s_sp.at[dst_i.at[j + 1]], add=True)
        return inner

    stage(0)
    pltpu.async_copy(gflat.at[src_i.at[0]], rows0, sem0)
    for b in range(RPT // CH):
        lax.fori_loop(0, CH // 2 - 1, pair, 0)
        pltpu.async_copy(gflat.at[src_i.at[CH - 1]], rows1, sem1)
        wait0()
        pltpu.sync_copy(rows0, s_sp.at[dst_i.at[CH - 2]], add=True)
        wait1()
        pltpu.sync_copy(rows1, s_sp.at[dst_i.at[CH - 1]], add=True)
        if b < RPT // CH - 1:
            stage(b + 1)
            pltpu.async_copy(gflat.at[src_i.at[0]], rows0, sem0)

    plsc.subcore_barrier()
    pltpu.sync_copy(s_sp.at[pl.ds(s * NPT, NPT)],
                    s_out.at[c, pl.ds(s * NPT, NPT)])


def _gmat_body(x_ref, w_ref, deg_ref, g_ref):
    deg = deg_ref[0][:, :1]
    dis = lax.rsqrt(deg + 1.0)
    h = jnp.dot(x_ref[0], w_ref[0], preferred_element_type=jnp.float32)
    g_ref[0] = h * dis


_gmat = pl.pallas_call(
    _gmat_body,
    grid=(NC, GB),
    in_specs=[
        pl.BlockSpec((1, NB, D), lambda g, i: (g, i, 0)),
        pl.BlockSpec((1, D, D), lambda g, i: (g, 0, 0)),
        pl.BlockSpec((1, NB, DL), lambda g, i: (g, i, 0)),
    ],
    out_specs=pl.BlockSpec((1, NB, D), lambda g, i: (g, i, 0)),
    out_shape=jax.ShapeDtypeStruct((NC, NP, D), jnp.float32),
    compiler_params=pltpu.CompilerParams(
        dimension_semantics=("arbitrary", "arbitrary")),
)


def _head_body(s_ref, deg_ref, b_ref, wp_ref, bp_ref, wf1_ref, bf1_ref,
               wf2_ref, bf2_ref, wo_ref, bo_ref, out_ref, acc_ref):
    i = pl.program_id(0)
    rem = N - i * NB
    mask = lax.broadcasted_iota(jnp.int32, (NB, D), 0) < rem
    reds = []
    for g in (0, 1):
        deg = deg_ref[g][:, :1]
        dis = lax.rsqrt(deg + 1.0)
        v = _leaky(dis * s_ref[g] + b_ref[g])
        v = jnp.where(mask, v, 0.0)
        reds.append(jnp.sum(v, axis=0, keepdims=True))

    @pl.when(i == 0)
    def _():
        acc_ref[0:1] = reds[0]
        acc_ref[1:2] = reds[1]

    @pl.when(i > 0)
    def _():
        acc_ref[0:1] += reds[0]
        acc_ref[1:2] += reds[1]

    @pl.when(i == GB - 1)
    def _():
        m1 = acc_ref[0:1] * (1.0 / N)
        m2 = acc_ref[1:2] * (1.0 / N)
        h1 = _leaky(jnp.dot(m1, wp_ref[0], preferred_element_type=jnp.float32)
                    + bp_ref[0])
        h2 = _leaky(jnp.dot(m2, wp_ref[1], preferred_element_type=jnp.float32)
                    + bp_ref[1])
        t = _leaky(jnp.dot(h1, wf1_ref[:D], preferred_element_type=jnp.float32)
                   + jnp.dot(h2, wf1_ref[D:], preferred_element_type=jnp.float32)
                   + bf1_ref[...])
        t = _leaky(jnp.dot(t, wf2_ref[...], preferred_element_type=jnp.float32)
                   + bf2_ref[...])
        o = jnp.dot(t, wo_ref[...], preferred_element_type=jnp.float32) + bo_ref[...]
        out_ref[...] = jax.nn.sigmoid(o)


_head = pl.pallas_call(
    _head_body,
    grid=(GB,),
    in_specs=[
        pl.BlockSpec((NC, NB, D), lambda i: (0, i, 0)),
        pl.BlockSpec((NC, NB, DL), lambda i: (0, i, 0)),
        pl.BlockSpec((NC, 1, D), lambda i: (0, 0, 0)),
        pl.BlockSpec((NC, D, D), lambda i: (0, 0, 0)),
        pl.BlockSpec((NC, 1, D), lambda i: (0, 0, 0)),
        pl.BlockSpec((2 * D, 256), lambda i: (0, 0)),
        pl.BlockSpec((1, 256), lambda i: (0, 0)),
        pl.BlockSpec((256, 64), lambda i: (0, 0)),
        pl.BlockSpec((1, 64), lambda i: (0, 0)),
        pl.BlockSpec((64, 1), lambda i: (0, 0)),
        pl.BlockSpec((1, 1), lambda i: (0, 0)),
    ],
    out_specs=pl.BlockSpec((1, 1), lambda i: (0, 0)),
    out_shape=jax.ShapeDtypeStruct((1, 1), jnp.float32),
    scratch_shapes=[pltpu.VMEM((2, D), jnp.float32)],
    compiler_params=pltpu.CompilerParams(
        dimension_semantics=("arbitrary",)),
)


def kernel(x1, edge_index1, x2, edge_index2, W1, b1, W2, b2, Wp1, bp1,
           Wp2, bp2, Wf1, bf1, Wf2, bf2, Wo, bo):
    pad = EP - E
    pad_src = (jnp.arange(pad, dtype=jnp.int32) % 128)
    pad_dst = N + (jnp.arange(pad, dtype=jnp.int32) % (NP - N))
    srcs = jnp.stack([
        jnp.concatenate([edge_index1[0], pad_src]),
        jnp.concatenate([edge_index2[0], pad_src]) + NP,
    ]).reshape(NC, ROWS, 128)
    dsts = jnp.stack([
        jnp.concatenate([edge_index1[1], pad_dst]),
        jnp.concatenate([edge_index2[1], pad_dst]),
    ]).reshape(NC, ROWS, 128)

    x_st = jnp.zeros((NC, NP, D), jnp.float32)
    x_st = x_st.at[0, :N].set(x1).at[1, :N].set(x2)
    w_st = jnp.stack([W1, W2])

    ones_hbm = jnp.ones((128, DL), jnp.float32)
    zdeg_hbm = jnp.zeros((NP, DL), jnp.float32)

    deg2d = _deg_kernel(dsts, ones_hbm, zdeg_hbm)
    gmat = _gmat(x_st, w_st, deg2d)
    smat = _scatter_kernel(gmat.reshape(NC * NP, D), srcs, dsts)

    b_st = jnp.stack([b1, b2]).reshape(NC, 1, D)
    wp_st = jnp.stack([Wp1, Wp2])
    bp_st = jnp.stack([bp1, bp2]).reshape(NC, 1, D)
    return _head(smat, deg2d, b_st, wp_st, bp_st, Wf1, bf1.reshape(1, 256),
                 Wf2, bf2.reshape(1, 64), Wo, bo.reshape(1, 1))
